# baseline (speedup 1.0000x reference)
import jax
import jax.numpy as jnp
from jax import lax
from jax.experimental import pallas as pl

NUM_USERS = 10000
NUM_ITEMS = 1000
NUM_CLASSES = 5
INPUT_DIM = 128
H0 = 64
H1 = 32
B = 1024

UB = 1000
NUB = NUM_USERS // UB


def _pass_a_body(ratings_ref, u_embT_ref, v_emb_ref, W1v_ref, b1v_ref,
                 W2vT_ref, msg_vT_ref, z_vT_ref):
    i = pl.program_id(0)

    @pl.when(i == 0)
    def _init():
        msg_vT_ref[...] = jnp.zeros_like(msg_vT_ref)
        z_vT_ref[...] = jnp.zeros_like(z_vT_ref)

    bf16 = jnp.bfloat16
    uT_blk = u_embT_ref[0]
    v_all = v_emb_ref[...]
    acc_h = jnp.zeros((UB, H0), dtype=jnp.float32)
    Rb = []
    for c in range(NUM_CLASSES):
        R = ratings_ref[c].astype(bf16)
        Rb.append(R)
        msgu = jnp.dot(R, v_all, preferred_element_type=jnp.float32)
        acc_h = acc_h + jnp.dot(msgu, W1v_ref[c],
                                preferred_element_type=jnp.float32)
        msg_vT_ref[c] += jnp.dot(uT_blk, R,
                                 preferred_element_type=jnp.float32)
    h_u = jnp.maximum(acc_h + b1v_ref[...], 0.0)
    h_uT = jnp.transpose(h_u)
    for c in range(NUM_CLASSES):
        pT = jnp.dot(W2vT_ref[c], h_uT,
                     preferred_element_type=jnp.float32).astype(bf16)
        z_vT_ref[...] += jnp.dot(pT, Rb[c],
                                 preferred_element_type=jnp.float32)


def _pass_g_body(msg_vT_ref, W1uT_ref, b1uT_ref, W2u_ref, G_ref):
    s = jnp.zeros((H0, NUM_ITEMS), dtype=jnp.float32)
    for c in range(NUM_CLASSES):
        s = s + jnp.dot(W1uT_ref[c], msg_vT_ref[c],
                        preferred_element_type=jnp.float32)
    h_vT = jnp.maximum(s + b1uT_ref[...], 0.0)
    h_v = jnp.transpose(h_vT)
    for c in range(NUM_CLASSES):
        G_ref[c] = jnp.dot(h_v, W2u_ref[c],
                           preferred_element_type=jnp.float32
                           ).astype(jnp.bfloat16)


def _pass_b_body(ratings_ref, G_ref, z_u_ref):
    acc = jnp.zeros((UB, H1), dtype=jnp.float32)
    for c in range(NUM_CLASSES):
        acc = acc + jnp.dot(ratings_ref[c].astype(jnp.bfloat16), G_ref[c],
                            preferred_element_type=jnp.float32)
    z_u_ref[...] = acc


def _pass_c_body(z_u_ref, z_vT_ref, u_ref, v_ref, n_ref, Q_ref,
                 out_ref, loss_ref, acc_ref):
    zu = jnp.zeros((B, H1), dtype=jnp.float32)
    for k in range(NUM_USERS // NUM_ITEMS):
        iota = lax.broadcasted_iota(jnp.int32, (B, NUM_ITEMS), 1) \
            + k * NUM_ITEMS
        oh = (u_ref[...] == iota).astype(jnp.float32)
        zu = zu + jnp.dot(oh, z_u_ref[pl.ds(k * NUM_ITEMS, NUM_ITEMS), :],
                          preferred_element_type=jnp.float32)
    z_v = jnp.transpose(z_vT_ref[...])
    iota_v = lax.broadcasted_iota(jnp.int32, (B, NUM_ITEMS), 1)
    ohv = (v_ref[...] == iota_v).astype(jnp.float32)
    zv = jnp.dot(ohv, z_v, preferred_element_type=jnp.float32)

    cols = []
    for c in range(NUM_CLASSES):
        t = jnp.dot(zu, Q_ref[c], preferred_element_type=jnp.float32)
        cols.append(jnp.sum(t * zv, axis=1, keepdims=True))
    logits = jnp.concatenate(cols, axis=1)
    out_ref[...] = logits

    m = jnp.max(logits, axis=1, keepdims=True)
    e = jnp.exp(logits - m)
    s = jnp.sum(e, axis=1, keepdims=True)
    logp = (logits - m) - jnp.log(s)
    iota_c = lax.broadcasted_iota(jnp.int32, (B, NUM_CLASSES), 1)
    lab = (n_ref[...] == iota_c).astype(jnp.float32)
    loss_ref[...] = jnp.reshape(-jnp.mean(jnp.sum(lab * logp, axis=1)), (1, 1))
    cand = jnp.where(logits == m, iota_c, NUM_CLASSES)
    am = jnp.min(cand, axis=1, keepdims=True)
    acc_ref[...] = jnp.reshape(
        jnp.mean((am == n_ref[...]).astype(jnp.float32)), (1, 1))


def kernel(u, v, n, ratings, u_emb, v_emb, W1u, b1u, W1v, b1v, W2u, W2v, Q):
    f32 = jnp.float32
    bf16 = jnp.bfloat16
    u_embT = jnp.swapaxes(u_emb.reshape(NUB, UB, INPUT_DIM), 1, 2) \
        .astype(bf16)
    v_emb_b = v_emb.astype(bf16)
    W2vT = jnp.swapaxes(W2v, 1, 2)
    W1uT = jnp.swapaxes(W1u, 1, 2)
    b1v2 = b1v.reshape(1, H0)
    b1uT = b1u.reshape(H0, 1)
    u2 = u.astype(jnp.int32).reshape(B, 1)
    v2 = v.astype(jnp.int32).reshape(B, 1)
    n2 = n.astype(jnp.int32).reshape(B, 1)

    msg_vT, z_vT = pl.pallas_call(
        _pass_a_body,
        grid=(NUB,),
        in_specs=[
            pl.BlockSpec((NUM_CLASSES, UB, NUM_ITEMS), lambda i: (0, i, 0)),
            pl.BlockSpec((1, INPUT_DIM, UB), lambda i: (i, 0, 0)),
            pl.BlockSpec((NUM_ITEMS, INPUT_DIM), lambda i: (0, 0)),
            pl.BlockSpec((NUM_CLASSES, INPUT_DIM, H0), lambda i: (0, 0, 0)),
            pl.BlockSpec((1, H0), lambda i: (0, 0)),
            pl.BlockSpec((NUM_CLASSES, H1, H0), lambda i: (0, 0, 0)),
        ],
        out_specs=[
            pl.BlockSpec((NUM_CLASSES, INPUT_DIM, NUM_ITEMS),
                         lambda i: (0, 0, 0)),
            pl.BlockSpec((H1, NUM_ITEMS), lambda i: (0, 0)),
        ],
        out_shape=[
            jax.ShapeDtypeStruct((NUM_CLASSES, INPUT_DIM, NUM_ITEMS), f32),
            jax.ShapeDtypeStruct((H1, NUM_ITEMS), f32),
        ],
    )(ratings, u_embT, v_emb_b, W1v, b1v2, W2vT)

    G = pl.pallas_call(
        _pass_g_body,
        in_specs=[
            pl.BlockSpec((NUM_CLASSES, INPUT_DIM, NUM_ITEMS),
                         lambda: (0, 0, 0)),
            pl.BlockSpec((NUM_CLASSES, H0, INPUT_DIM), lambda: (0, 0, 0)),
            pl.BlockSpec((H0, 1), lambda: (0, 0)),
            pl.BlockSpec((NUM_CLASSES, H0, H1), lambda: (0, 0, 0)),
        ],
        out_specs=pl.BlockSpec((NUM_CLASSES, NUM_ITEMS, H1), lambda: (0, 0, 0)),
        out_shape=jax.ShapeDtypeStruct((NUM_CLASSES, NUM_ITEMS, H1), bf16),
    )(msg_vT, W1uT, b1uT, W2u)

    z_u = pl.pallas_call(
        _pass_b_body,
        grid=(NUB,),
        in_specs=[
            pl.BlockSpec((NUM_CLASSES, UB, NUM_ITEMS), lambda i: (0, i, 0)),
            pl.BlockSpec((NUM_CLASSES, NUM_ITEMS, H1), lambda i: (0, 0, 0)),
        ],
        out_specs=pl.BlockSpec((UB, H1), lambda i: (i, 0)),
        out_shape=jax.ShapeDtypeStruct((NUM_USERS, H1), f32),
    )(ratings, G)

    outputs, loss, accuracy = pl.pallas_call(
        _pass_c_body,
        in_specs=[
            pl.BlockSpec((NUM_USERS, H1), lambda: (0, 0)),
            pl.BlockSpec((H1, NUM_ITEMS), lambda: (0, 0)),
            pl.BlockSpec((B, 1), lambda: (0, 0)),
            pl.BlockSpec((B, 1), lambda: (0, 0)),
            pl.BlockSpec((B, 1), lambda: (0, 0)),
            pl.BlockSpec((NUM_CLASSES, H1, H1), lambda: (0, 0, 0)),
        ],
        out_specs=[
            pl.BlockSpec((B, NUM_CLASSES), lambda: (0, 0)),
            pl.BlockSpec((1, 1), lambda: (0, 0)),
            pl.BlockSpec((1, 1), lambda: (0, 0)),
        ],
        out_shape=[
            jax.ShapeDtypeStruct((B, NUM_CLASSES), f32),
            jax.ShapeDtypeStruct((1, 1), f32),
            jax.ShapeDtypeStruct((1, 1), f32),
        ],
    )(z_u, z_vT, u2, v2, n2, Q)

    return (outputs, loss.reshape(()), accuracy.reshape(()))


# speedup vs baseline: 1.0189x; 1.0189x over previous
import jax
import jax.numpy as jnp
from jax import lax
from jax.experimental import pallas as pl

NUM_USERS = 10000
NUM_ITEMS = 1000
NUM_CLASSES = 5
INPUT_DIM = 128
H0 = 64
H1 = 32
B = 1024

UB = 1000
NUB = NUM_USERS // UB


def _pass_a_body(r0, r1, r2, r3, r4, u_embT_ref, v_emb_ref, W1v_ref, b1v_ref,
                 W2vT_ref, msg_vT_ref, z_vT_ref):
    rating_refs = (r0, r1, r2, r3, r4)
    i = pl.program_id(0)

    @pl.when(i == 0)
    def _init():
        msg_vT_ref[...] = jnp.zeros_like(msg_vT_ref)
        z_vT_ref[...] = jnp.zeros_like(z_vT_ref)

    bf16 = jnp.bfloat16
    uT_blk = u_embT_ref[0]
    v_all = v_emb_ref[...]
    acc_h = jnp.zeros((UB, H0), dtype=jnp.float32)
    Rb = []
    for c in range(NUM_CLASSES):
        R = rating_refs[c][0].astype(bf16)
        Rb.append(R)
        msgu = jnp.dot(R, v_all, preferred_element_type=jnp.float32)
        acc_h = acc_h + jnp.dot(msgu, W1v_ref[c],
                                preferred_element_type=jnp.float32)
        msg_vT_ref[c] += jnp.dot(uT_blk, R,
                                 preferred_element_type=jnp.float32)
    h_u = jnp.maximum(acc_h + b1v_ref[...], 0.0)
    h_uT = jnp.transpose(h_u)
    for c in range(NUM_CLASSES):
        pT = jnp.dot(W2vT_ref[c], h_uT,
                     preferred_element_type=jnp.float32).astype(bf16)
        z_vT_ref[...] += jnp.dot(pT, Rb[c],
                                 preferred_element_type=jnp.float32)


def _pass_g_body(msg_vT_ref, W1uT_ref, b1uT_ref, W2u_ref, G_ref):
    s = jnp.zeros((H0, NUM_ITEMS), dtype=jnp.float32)
    for c in range(NUM_CLASSES):
        s = s + jnp.dot(W1uT_ref[c], msg_vT_ref[c],
                        preferred_element_type=jnp.float32)
    h_vT = jnp.maximum(s + b1uT_ref[...], 0.0)
    h_v = jnp.transpose(h_vT)
    for c in range(NUM_CLASSES):
        G_ref[c] = jnp.dot(h_v, W2u_ref[c],
                           preferred_element_type=jnp.float32
                           ).astype(jnp.bfloat16)


def _pass_b_body(r0, r1, r2, r3, r4, G_ref, z_u_ref):
    rating_refs = (r0, r1, r2, r3, r4)
    acc = jnp.zeros((UB, H1), dtype=jnp.float32)
    for c in range(NUM_CLASSES):
        acc = acc + jnp.dot(rating_refs[c][0].astype(jnp.bfloat16), G_ref[c],
                            preferred_element_type=jnp.float32)
    z_u_ref[...] = acc


def _pass_c_body(z_u_ref, z_vT_ref, u_ref, v_ref, n_ref, Q_ref,
                 out_ref, loss_ref, acc_ref):
    zu = jnp.zeros((B, H1), dtype=jnp.float32)
    for k in range(NUM_USERS // NUM_ITEMS):
        iota = lax.broadcasted_iota(jnp.int32, (B, NUM_ITEMS), 1) \
            + k * NUM_ITEMS
        oh = (u_ref[...] == iota).astype(jnp.float32)
        zu = zu + jnp.dot(oh, z_u_ref[pl.ds(k * NUM_ITEMS, NUM_ITEMS), :],
                          preferred_element_type=jnp.float32)
    z_v = jnp.transpose(z_vT_ref[...])
    iota_v = lax.broadcasted_iota(jnp.int32, (B, NUM_ITEMS), 1)
    ohv = (v_ref[...] == iota_v).astype(jnp.float32)
    zv = jnp.dot(ohv, z_v, preferred_element_type=jnp.float32)

    cols = []
    for c in range(NUM_CLASSES):
        t = jnp.dot(zu, Q_ref[c], preferred_element_type=jnp.float32)
        cols.append(jnp.sum(t * zv, axis=1, keepdims=True))
    logits = jnp.concatenate(cols, axis=1)
    out_ref[...] = logits

    m = jnp.max(logits, axis=1, keepdims=True)
    e = jnp.exp(logits - m)
    s = jnp.sum(e, axis=1, keepdims=True)
    logp = (logits - m) - jnp.log(s)
    iota_c = lax.broadcasted_iota(jnp.int32, (B, NUM_CLASSES), 1)
    lab = (n_ref[...] == iota_c).astype(jnp.float32)
    loss_ref[...] = jnp.reshape(-jnp.mean(jnp.sum(lab * logp, axis=1)), (1, 1))
    cand = jnp.where(logits == m, iota_c, NUM_CLASSES)
    am = jnp.min(cand, axis=1, keepdims=True)
    acc_ref[...] = jnp.reshape(
        jnp.mean((am == n_ref[...]).astype(jnp.float32)), (1, 1))


def kernel(u, v, n, ratings, u_emb, v_emb, W1u, b1u, W1v, b1v, W2u, W2v, Q):
    f32 = jnp.float32
    bf16 = jnp.bfloat16
    u_embT = jnp.swapaxes(u_emb.reshape(NUB, UB, INPUT_DIM), 1, 2) \
        .astype(bf16)
    v_emb_b = v_emb.astype(bf16)
    W2vT = jnp.swapaxes(W2v, 1, 2)
    W1uT = jnp.swapaxes(W1u, 1, 2)
    b1v2 = b1v.reshape(1, H0)
    b1uT = b1u.reshape(H0, 1)
    u2 = u.astype(jnp.int32).reshape(B, 1)
    v2 = v.astype(jnp.int32).reshape(B, 1)
    n2 = n.astype(jnp.int32).reshape(B, 1)

    msg_vT, z_vT = pl.pallas_call(
        _pass_a_body,
        grid=(NUB,),
        in_specs=[
            *[pl.BlockSpec((1, UB, NUM_ITEMS), lambda i, c=c: (c, i, 0))
              for c in range(NUM_CLASSES)],
            pl.BlockSpec((1, INPUT_DIM, UB), lambda i: (i, 0, 0)),
            pl.BlockSpec((NUM_ITEMS, INPUT_DIM), lambda i: (0, 0)),
            pl.BlockSpec((NUM_CLASSES, INPUT_DIM, H0), lambda i: (0, 0, 0)),
            pl.BlockSpec((1, H0), lambda i: (0, 0)),
            pl.BlockSpec((NUM_CLASSES, H1, H0), lambda i: (0, 0, 0)),
        ],
        out_specs=[
            pl.BlockSpec((NUM_CLASSES, INPUT_DIM, NUM_ITEMS),
                         lambda i: (0, 0, 0)),
            pl.BlockSpec((H1, NUM_ITEMS), lambda i: (0, 0)),
        ],
        out_shape=[
            jax.ShapeDtypeStruct((NUM_CLASSES, INPUT_DIM, NUM_ITEMS), f32),
            jax.ShapeDtypeStruct((H1, NUM_ITEMS), f32),
        ],
    )(ratings, ratings, ratings, ratings, ratings,
      u_embT, v_emb_b, W1v, b1v2, W2vT)

    G = pl.pallas_call(
        _pass_g_body,
        in_specs=[
            pl.BlockSpec((NUM_CLASSES, INPUT_DIM, NUM_ITEMS),
                         lambda: (0, 0, 0)),
            pl.BlockSpec((NUM_CLASSES, H0, INPUT_DIM), lambda: (0, 0, 0)),
            pl.BlockSpec((H0, 1), lambda: (0, 0)),
            pl.BlockSpec((NUM_CLASSES, H0, H1), lambda: (0, 0, 0)),
        ],
        out_specs=pl.BlockSpec((NUM_CLASSES, NUM_ITEMS, H1), lambda: (0, 0, 0)),
        out_shape=jax.ShapeDtypeStruct((NUM_CLASSES, NUM_ITEMS, H1), bf16),
    )(msg_vT, W1uT, b1uT, W2u)

    z_u = pl.pallas_call(
        _pass_b_body,
        grid=(NUB,),
        in_specs=[
            *[pl.BlockSpec((1, UB, NUM_ITEMS), lambda i, c=c: (c, i, 0))
              for c in range(NUM_CLASSES)],
            pl.BlockSpec((NUM_CLASSES, NUM_ITEMS, H1), lambda i: (0, 0, 0)),
        ],
        out_specs=pl.BlockSpec((UB, H1), lambda i: (i, 0)),
        out_shape=jax.ShapeDtypeStruct((NUM_USERS, H1), f32),
    )(ratings, ratings, ratings, ratings, ratings, G)

    outputs, loss, accuracy = pl.pallas_call(
        _pass_c_body,
        in_specs=[
            pl.BlockSpec((NUM_USERS, H1), lambda: (0, 0)),
            pl.BlockSpec((H1, NUM_ITEMS), lambda: (0, 0)),
            pl.BlockSpec((B, 1), lambda: (0, 0)),
            pl.BlockSpec((B, 1), lambda: (0, 0)),
            pl.BlockSpec((B, 1), lambda: (0, 0)),
            pl.BlockSpec((NUM_CLASSES, H1, H1), lambda: (0, 0, 0)),
        ],
        out_specs=[
            pl.BlockSpec((B, NUM_CLASSES), lambda: (0, 0)),
            pl.BlockSpec((1, 1), lambda: (0, 0)),
            pl.BlockSpec((1, 1), lambda: (0, 0)),
        ],
        out_shape=[
            jax.ShapeDtypeStruct((B, NUM_CLASSES), f32),
            jax.ShapeDtypeStruct((1, 1), f32),
            jax.ShapeDtypeStruct((1, 1), f32),
        ],
    )(z_u, z_vT, u2, v2, n2, Q)

    return (outputs, loss.reshape(()), accuracy.reshape(()))


# speedup vs baseline: 1.4587x; 1.4316x over previous
import jax
import jax.numpy as jnp
from jax import lax
from jax.experimental import pallas as pl

NUM_USERS = 10000
NUM_ITEMS = 1000
NUM_CLASSES = 5
INPUT_DIM = 128
H0 = 64
H1 = 32
B = 1024

VB = 40
NBV = NUM_ITEMS // VB


def _prep_body(v_emb_ref, W1v_ref, E_ref):
    ve = v_emb_ref[...]
    for c in range(NUM_CLASSES):
        t = jnp.dot(ve, W1v_ref[c], preferred_element_type=jnp.float32)
        Ec = jnp.transpose(t).astype(jnp.bfloat16)
        for j in range(NBV):
            E_ref[j, c] = Ec[:, j * VB:(j + 1) * VB]


def _pass_a_body(r0, r1, r2, r3, r4, u_emb_ref, W1u_ref, b1u_ref, W2uT_ref,
                 E_ref, z_uT_ref, acc_hT_ref):
    rating_refs = (r0, r1, r2, r3, r4)
    i = pl.program_id(0)

    @pl.when(i == 0)
    def _init():
        z_uT_ref[...] = jnp.zeros_like(z_uT_ref)
        acc_hT_ref[...] = jnp.zeros_like(acc_hT_ref)

    bf16 = jnp.bfloat16
    u_all = u_emb_ref[...]
    s = jnp.zeros((VB, H0), dtype=jnp.float32)
    for c in range(NUM_CLASSES):
        Rt = rating_refs[c][0].astype(bf16)
        msgv = jnp.dot(Rt, u_all, preferred_element_type=jnp.float32)
        s = s + jnp.dot(msgv, W1u_ref[c],
                        preferred_element_type=jnp.float32)
        acc_hT_ref[...] += jnp.dot(E_ref[0, c], Rt,
                                   preferred_element_type=jnp.float32)
    h_v = jnp.maximum(s + b1u_ref[...], 0.0)
    h_vT = jnp.transpose(h_v)
    for c in range(NUM_CLASSES):
        Rt = rating_refs[c][0].astype(bf16)
        GcT = jnp.dot(W2uT_ref[c], h_vT,
                      preferred_element_type=jnp.float32).astype(bf16)
        z_uT_ref[...] += jnp.dot(GcT, Rt,
                                 preferred_element_type=jnp.float32)


def _mid_body(acc_hT_ref, b1vT_ref, W2vT_ref, z_uT_ref, p_ref, z_u_ref):
    h_uT = jnp.maximum(acc_hT_ref[...] + b1vT_ref[...], 0.0)
    for c in range(NUM_CLASSES):
        pcT = jnp.dot(W2vT_ref[c], h_uT,
                      preferred_element_type=jnp.float32)
        p_ref[c] = jnp.transpose(pcT).astype(jnp.bfloat16)
    z_u_ref[...] = jnp.transpose(z_uT_ref[...])


def _pass_b_body(r0, r1, r2, r3, r4, p_ref, z_v_ref):
    rating_refs = (r0, r1, r2, r3, r4)
    acc = jnp.zeros((VB, H1), dtype=jnp.float32)
    for c in range(NUM_CLASSES):
        acc = acc + jnp.dot(rating_refs[c][0].astype(jnp.bfloat16), p_ref[c],
                            preferred_element_type=jnp.float32)
    z_v_ref[...] = acc


def _pass_c_body(z_u_ref, z_v_ref, u_ref, v_ref, n_ref, Q_ref,
                 out_ref, loss_ref, acc_ref):
    zu = jnp.zeros((B, H1), dtype=jnp.float32)
    for k in range(NUM_USERS // NUM_ITEMS):
        iota = lax.broadcasted_iota(jnp.int32, (B, NUM_ITEMS), 1) \
            + k * NUM_ITEMS
        oh = (u_ref[...] == iota).astype(jnp.float32)
        zu = zu + jnp.dot(oh, z_u_ref[pl.ds(k * NUM_ITEMS, NUM_ITEMS), :],
                          preferred_element_type=jnp.float32)
    iota_v = lax.broadcasted_iota(jnp.int32, (B, NUM_ITEMS), 1)
    ohv = (v_ref[...] == iota_v).astype(jnp.float32)
    zv = jnp.dot(ohv, z_v_ref[...], preferred_element_type=jnp.float32)

    cols = []
    for c in range(NUM_CLASSES):
        t = jnp.dot(zu, Q_ref[c], preferred_element_type=jnp.float32)
        cols.append(jnp.sum(t * zv, axis=1, keepdims=True))
    logits = jnp.concatenate(cols, axis=1)
    out_ref[...] = logits

    m = jnp.max(logits, axis=1, keepdims=True)
    e = jnp.exp(logits - m)
    s = jnp.sum(e, axis=1, keepdims=True)
    logp = (logits - m) - jnp.log(s)
    iota_c = lax.broadcasted_iota(jnp.int32, (B, NUM_CLASSES), 1)
    lab = (n_ref[...] == iota_c).astype(jnp.float32)
    loss_ref[...] = jnp.reshape(-jnp.mean(jnp.sum(lab * logp, axis=1)), (1, 1))
    cand = jnp.where(logits == m, iota_c, NUM_CLASSES)
    am = jnp.min(cand, axis=1, keepdims=True)
    acc_ref[...] = jnp.reshape(
        jnp.mean((am == n_ref[...]).astype(jnp.float32)), (1, 1))


def kernel(u, v, n, ratings, u_emb, v_emb, W1u, b1u, W1v, b1v, W2u, W2v, Q):
    f32 = jnp.float32
    bf16 = jnp.bfloat16
    rT = jnp.swapaxes(ratings, 1, 2)
    u_emb_b = u_emb.astype(bf16)
    W2uT = jnp.swapaxes(W2u, 1, 2)
    W2vT = jnp.swapaxes(W2v, 1, 2)
    b1u2 = b1u.reshape(1, H0)
    b1vT = b1v.reshape(H0, 1)
    u2 = u.astype(jnp.int32).reshape(B, 1)
    v2 = v.astype(jnp.int32).reshape(B, 1)
    n2 = n.astype(jnp.int32).reshape(B, 1)

    E = pl.pallas_call(
        _prep_body,
        in_specs=[
            pl.BlockSpec((NUM_ITEMS, INPUT_DIM), lambda: (0, 0)),
            pl.BlockSpec((NUM_CLASSES, INPUT_DIM, H0), lambda: (0, 0, 0)),
        ],
        out_specs=pl.BlockSpec((NBV, NUM_CLASSES, H0, VB),
                               lambda: (0, 0, 0, 0)),
        out_shape=jax.ShapeDtypeStruct((NBV, NUM_CLASSES, H0, VB), bf16),
    )(v_emb, W1v)

    z_uT, acc_hT = pl.pallas_call(
        _pass_a_body,
        grid=(NBV,),
        in_specs=[
            *[pl.BlockSpec((1, VB, NUM_USERS), lambda i, c=c: (c, i, 0))
              for c in range(NUM_CLASSES)],
            pl.BlockSpec((NUM_USERS, INPUT_DIM), lambda i: (0, 0)),
            pl.BlockSpec((NUM_CLASSES, INPUT_DIM, H0), lambda i: (0, 0, 0)),
            pl.BlockSpec((1, H0), lambda i: (0, 0)),
            pl.BlockSpec((NUM_CLASSES, H1, H0), lambda i: (0, 0, 0)),
            pl.BlockSpec((1, NUM_CLASSES, H0, VB), lambda i: (i, 0, 0, 0)),
        ],
        out_specs=[
            pl.BlockSpec((H1, NUM_USERS), lambda i: (0, 0)),
            pl.BlockSpec((H0, NUM_USERS), lambda i: (0, 0)),
        ],
        out_shape=[
            jax.ShapeDtypeStruct((H1, NUM_USERS), f32),
            jax.ShapeDtypeStruct((H0, NUM_USERS), f32),
        ],
    )(rT, rT, rT, rT, rT, u_emb_b, W1u, b1u2, W2uT, E)

    p, z_u = pl.pallas_call(
        _mid_body,
        in_specs=[
            pl.BlockSpec((H0, NUM_USERS), lambda: (0, 0)),
            pl.BlockSpec((H0, 1), lambda: (0, 0)),
            pl.BlockSpec((NUM_CLASSES, H1, H0), lambda: (0, 0, 0)),
            pl.BlockSpec((H1, NUM_USERS), lambda: (0, 0)),
        ],
        out_specs=[
            pl.BlockSpec((NUM_CLASSES, NUM_USERS, H1), lambda: (0, 0, 0)),
            pl.BlockSpec((NUM_USERS, H1), lambda: (0, 0)),
        ],
        out_shape=[
            jax.ShapeDtypeStruct((NUM_CLASSES, NUM_USERS, H1), bf16),
            jax.ShapeDtypeStruct((NUM_USERS, H1), f32),
        ],
    )(acc_hT, b1vT, W2vT, z_uT)

    z_v = pl.pallas_call(
        _pass_b_body,
        grid=(NBV,),
        in_specs=[
            *[pl.BlockSpec((1, VB, NUM_USERS), lambda i, c=c: (c, i, 0))
              for c in range(NUM_CLASSES)],
            pl.BlockSpec((NUM_CLASSES, NUM_USERS, H1), lambda i: (0, 0, 0)),
        ],
        out_specs=pl.BlockSpec((VB, H1), lambda i: (i, 0)),
        out_shape=jax.ShapeDtypeStruct((NUM_ITEMS, H1), f32),
    )(rT, rT, rT, rT, rT, p)

    outputs, loss, accuracy = pl.pallas_call(
        _pass_c_body,
        in_specs=[
            pl.BlockSpec((NUM_USERS, H1), lambda: (0, 0)),
            pl.BlockSpec((NUM_ITEMS, H1), lambda: (0, 0)),
            pl.BlockSpec((B, 1), lambda: (0, 0)),
            pl.BlockSpec((B, 1), lambda: (0, 0)),
            pl.BlockSpec((B, 1), lambda: (0, 0)),
            pl.BlockSpec((NUM_CLASSES, H1, H1), lambda: (0, 0, 0)),
        ],
        out_specs=[
            pl.BlockSpec((B, NUM_CLASSES), lambda: (0, 0)),
            pl.BlockSpec((1, 1), lambda: (0, 0)),
            pl.BlockSpec((1, 1), lambda: (0, 0)),
        ],
        out_shape=[
            jax.ShapeDtypeStruct((B, NUM_CLASSES), f32),
            jax.ShapeDtypeStruct((1, 1), f32),
            jax.ShapeDtypeStruct((1, 1), f32),
        ],
    )(z_u, z_v, u2, v2, n2, Q)

    return (outputs, loss.reshape(()), accuracy.reshape(()))


# speedup vs baseline: 1.9024x; 1.3042x over previous
import jax
import jax.numpy as jnp
from jax import lax
from jax.experimental import pallas as pl

NUM_USERS = 10000
NUM_ITEMS = 1000
NUM_CLASSES = 5
INPUT_DIM = 128
H0 = 64
H1 = 32
B = 1024

VB = 40
NBV = NUM_ITEMS // VB


def _prep_body(v_emb_ref, W1v_ref, E_ref):
    ve = v_emb_ref[...]
    for c in range(NUM_CLASSES):
        t = jnp.dot(ve, W1v_ref[c], preferred_element_type=jnp.float32)
        Ec = jnp.transpose(t).astype(jnp.bfloat16)
        for j in range(NBV):
            E_ref[j, :, c * VB:(c + 1) * VB] = Ec[:, j * VB:(j + 1) * VB]


def _pass_a_body(r0, r1, r2, r3, r4, u_emb_ref, W1u_ref, b1u_ref, W2uT_ref,
                 E_ref, z_uT_ref, acc_hT_ref):
    rating_refs = (r0, r1, r2, r3, r4)
    i = pl.program_id(0)

    @pl.when(i == 0)
    def _init():
        z_uT_ref[...] = jnp.zeros_like(z_uT_ref)
        acc_hT_ref[...] = jnp.zeros_like(acc_hT_ref)

    bf16 = jnp.bfloat16
    u_all = u_emb_ref[...]
    Rtcat = jnp.concatenate(
        [rating_refs[c][0] for c in range(NUM_CLASSES)],
        axis=0).astype(bf16)
    msgv_all = jnp.dot(Rtcat, u_all,
                       preferred_element_type=jnp.float32)
    s = jnp.zeros((VB, H0), dtype=jnp.float32)
    for c in range(NUM_CLASSES):
        s = s + jnp.dot(msgv_all[c * VB:(c + 1) * VB], W1u_ref[c],
                        preferred_element_type=jnp.float32)
    acc_hT_ref[...] += jnp.dot(E_ref[0], Rtcat,
                               preferred_element_type=jnp.float32)
    h_v = jnp.maximum(s + b1u_ref[...], 0.0)
    h_vT = jnp.transpose(h_v)
    Gcat = jnp.concatenate(
        [jnp.dot(W2uT_ref[c], h_vT, preferred_element_type=jnp.float32)
         for c in range(NUM_CLASSES)], axis=1).astype(bf16)
    z_uT_ref[...] += jnp.dot(Gcat, Rtcat,
                             preferred_element_type=jnp.float32)


def _mid_body(acc_hT_ref, b1vT_ref, W2vT_ref, z_uT_ref, p_ref, z_u_ref):
    h_uT = jnp.maximum(acc_hT_ref[...] + b1vT_ref[...], 0.0)
    for c in range(NUM_CLASSES):
        pcT = jnp.dot(W2vT_ref[c], h_uT,
                      preferred_element_type=jnp.float32)
        p_ref[c] = jnp.transpose(pcT).astype(jnp.bfloat16)
    z_u_ref[...] = jnp.transpose(z_uT_ref[...])


def _pass_b_body(r0, r1, r2, r3, r4, p_ref, z_v_ref):
    rating_refs = (r0, r1, r2, r3, r4)
    acc = jnp.zeros((VB, H1), dtype=jnp.float32)
    for c in range(NUM_CLASSES):
        acc = acc + jnp.dot(rating_refs[c][0].astype(jnp.bfloat16), p_ref[c],
                            preferred_element_type=jnp.float32)
    z_v_ref[...] = acc


def _pass_c_body(z_u_ref, z_v_ref, u_ref, v_ref, n_ref, Q_ref,
                 out_ref, loss_ref, acc_ref):
    zu = jnp.zeros((B, H1), dtype=jnp.float32)
    for k in range(NUM_USERS // NUM_ITEMS):
        iota = lax.broadcasted_iota(jnp.int32, (B, NUM_ITEMS), 1) \
            + k * NUM_ITEMS
        oh = (u_ref[...] == iota).astype(jnp.float32)
        zu = zu + jnp.dot(oh, z_u_ref[pl.ds(k * NUM_ITEMS, NUM_ITEMS), :],
                          preferred_element_type=jnp.float32)
    iota_v = lax.broadcasted_iota(jnp.int32, (B, NUM_ITEMS), 1)
    ohv = (v_ref[...] == iota_v).astype(jnp.float32)
    zv = jnp.dot(ohv, z_v_ref[...], preferred_element_type=jnp.float32)

    cols = []
    for c in range(NUM_CLASSES):
        t = jnp.dot(zu, Q_ref[c], preferred_element_type=jnp.float32)
        cols.append(jnp.sum(t * zv, axis=1, keepdims=True))
    logits = jnp.concatenate(cols, axis=1)
    out_ref[...] = logits

    m = jnp.max(logits, axis=1, keepdims=True)
    e = jnp.exp(logits - m)
    s = jnp.sum(e, axis=1, keepdims=True)
    logp = (logits - m) - jnp.log(s)
    iota_c = lax.broadcasted_iota(jnp.int32, (B, NUM_CLASSES), 1)
    lab = (n_ref[...] == iota_c).astype(jnp.float32)
    loss_ref[...] = jnp.reshape(-jnp.mean(jnp.sum(lab * logp, axis=1)), (1, 1))
    cand = jnp.where(logits == m, iota_c, NUM_CLASSES)
    am = jnp.min(cand, axis=1, keepdims=True)
    acc_ref[...] = jnp.reshape(
        jnp.mean((am == n_ref[...]).astype(jnp.float32)), (1, 1))


def kernel(u, v, n, ratings, u_emb, v_emb, W1u, b1u, W1v, b1v, W2u, W2v, Q):
    f32 = jnp.float32
    bf16 = jnp.bfloat16
    rT = jnp.swapaxes(ratings, 1, 2)
    u_emb_b = u_emb.astype(bf16)
    W2uT = jnp.swapaxes(W2u, 1, 2)
    W2vT = jnp.swapaxes(W2v, 1, 2)
    b1u2 = b1u.reshape(1, H0)
    b1vT = b1v.reshape(H0, 1)
    u2 = u.astype(jnp.int32).reshape(B, 1)
    v2 = v.astype(jnp.int32).reshape(B, 1)
    n2 = n.astype(jnp.int32).reshape(B, 1)

    E = pl.pallas_call(
        _prep_body,
        in_specs=[
            pl.BlockSpec((NUM_ITEMS, INPUT_DIM), lambda: (0, 0)),
            pl.BlockSpec((NUM_CLASSES, INPUT_DIM, H0), lambda: (0, 0, 0)),
        ],
        out_specs=pl.BlockSpec((NBV, H0, NUM_CLASSES * VB),
                               lambda: (0, 0, 0)),
        out_shape=jax.ShapeDtypeStruct((NBV, H0, NUM_CLASSES * VB), bf16),
    )(v_emb, W1v)

    z_uT, acc_hT = pl.pallas_call(
        _pass_a_body,
        grid=(NBV,),
        in_specs=[
            *[pl.BlockSpec((1, VB, NUM_USERS), lambda i, c=c: (c, i, 0))
              for c in range(NUM_CLASSES)],
            pl.BlockSpec((NUM_USERS, INPUT_DIM), lambda i: (0, 0)),
            pl.BlockSpec((NUM_CLASSES, INPUT_DIM, H0), lambda i: (0, 0, 0)),
            pl.BlockSpec((1, H0), lambda i: (0, 0)),
            pl.BlockSpec((NUM_CLASSES, H1, H0), lambda i: (0, 0, 0)),
            pl.BlockSpec((1, H0, NUM_CLASSES * VB), lambda i: (i, 0, 0)),
        ],
        out_specs=[
            pl.BlockSpec((H1, NUM_USERS), lambda i: (0, 0)),
            pl.BlockSpec((H0, NUM_USERS), lambda i: (0, 0)),
        ],
        out_shape=[
            jax.ShapeDtypeStruct((H1, NUM_USERS), f32),
            jax.ShapeDtypeStruct((H0, NUM_USERS), f32),
        ],
    )(rT, rT, rT, rT, rT, u_emb_b, W1u, b1u2, W2uT, E)

    p, z_u = pl.pallas_call(
        _mid_body,
        in_specs=[
            pl.BlockSpec((H0, NUM_USERS), lambda: (0, 0)),
            pl.BlockSpec((H0, 1), lambda: (0, 0)),
            pl.BlockSpec((NUM_CLASSES, H1, H0), lambda: (0, 0, 0)),
            pl.BlockSpec((H1, NUM_USERS), lambda: (0, 0)),
        ],
        out_specs=[
            pl.BlockSpec((NUM_CLASSES, NUM_USERS, H1), lambda: (0, 0, 0)),
            pl.BlockSpec((NUM_USERS, H1), lambda: (0, 0)),
        ],
        out_shape=[
            jax.ShapeDtypeStruct((NUM_CLASSES, NUM_USERS, H1), bf16),
            jax.ShapeDtypeStruct((NUM_USERS, H1), f32),
        ],
    )(acc_hT, b1vT, W2vT, z_uT)

    z_v = pl.pallas_call(
        _pass_b_body,
        grid=(NBV,),
        in_specs=[
            *[pl.BlockSpec((1, VB, NUM_USERS), lambda i, c=c: (c, i, 0))
              for c in range(NUM_CLASSES)],
            pl.BlockSpec((NUM_CLASSES, NUM_USERS, H1), lambda i: (0, 0, 0)),
        ],
        out_specs=pl.BlockSpec((VB, H1), lambda i: (i, 0)),
        out_shape=jax.ShapeDtypeStruct((NUM_ITEMS, H1), f32),
    )(rT, rT, rT, rT, rT, p)

    outputs, loss, accuracy = pl.pallas_call(
        _pass_c_body,
        in_specs=[
            pl.BlockSpec((NUM_USERS, H1), lambda: (0, 0)),
            pl.BlockSpec((NUM_ITEMS, H1), lambda: (0, 0)),
            pl.BlockSpec((B, 1), lambda: (0, 0)),
            pl.BlockSpec((B, 1), lambda: (0, 0)),
            pl.BlockSpec((B, 1), lambda: (0, 0)),
            pl.BlockSpec((NUM_CLASSES, H1, H1), lambda: (0, 0, 0)),
        ],
        out_specs=[
            pl.BlockSpec((B, NUM_CLASSES), lambda: (0, 0)),
            pl.BlockSpec((1, 1), lambda: (0, 0)),
            pl.BlockSpec((1, 1), lambda: (0, 0)),
        ],
        out_shape=[
            jax.ShapeDtypeStruct((B, NUM_CLASSES), f32),
            jax.ShapeDtypeStruct((1, 1), f32),
            jax.ShapeDtypeStruct((1, 1), f32),
        ],
    )(z_u, z_v, u2, v2, n2, Q)

    return (outputs, loss.reshape(()), accuracy.reshape(()))
